# Initial kernel scaffold; baseline (speedup 1.0000x reference)
import jax
import jax.numpy as jnp
from jax.experimental import pallas as pl


def kernel(word_idx, ctx_inds, ctx_lens, neg_inds, emb0_weight, emb1_weight):
    raise NotImplementedError("write your pallas kernel here")


# speedup vs baseline: 2.7581x; 2.7581x over previous
import jax
import jax.numpy as jnp
from jax import lax
from jax.experimental import pallas as pl
from jax.experimental.pallas import tpu as pltpu
from jax.experimental.pallas import tpu_sc as plsc

_VOCAB = 100000
_DIM = 64
_B = 16384
_L = 10
_NEG = 5
_T = _NEG + 1
_NC = 2
_NS = 16
_NW = _NC * _NS
_BPW = _B // _NW
_CH = 64
_NCHUNK = _BPW // _CH
_LANES = 16
_DC = _DIM // _LANES
_YROWS = _B * _T
_Y2COLS = 256
_Y2ROWS = _YROWS * _LANES // _Y2COLS


def _cbow_sc_body(ctx_idx_hbm, tgt_idx_hbm, lens_hbm, emb0_hbm, emb1_hbm,
                  y_hbm,
                  ctx_idx, tgt_idx, lens_v, ctx_rows, tgt_rows, y_v,
                  sem_c, sem_t):
    wid = lax.axis_index("s") * _NC + lax.axis_index("c")
    base = wid * _BPW

    def chunk_body(c, carry):
        cb = base + c * _CH
        pltpu.sync_copy(
            ctx_idx_hbm.at[pl.ds(pl.multiple_of(cb * _L, 8), _CH * _L)],
            ctx_idx)
        pltpu.sync_copy(
            tgt_idx_hbm.at[pl.ds(pl.multiple_of(cb * _T, 8), _CH * _T)],
            tgt_idx)
        pltpu.sync_copy(lens_hbm.at[pl.ds(pl.multiple_of(cb, 8), _CH)],
                        lens_v)
        handles = []
        for j in range(_CH * _L // 128):
            handles.append(pltpu.async_copy(
                emb0_hbm.at[ctx_idx.at[pl.ds(j * 128, 128)]],
                ctx_rows.at[pl.ds(j * 128, 128)], sem_c))
        for j in range(_CH * _T // 128):
            handles.append(pltpu.async_copy(
                emb1_hbm.at[tgt_idx.at[pl.ds(j * 128, 128)]],
                tgt_rows.at[pl.ds(j * 128, 128)], sem_t))
        for h in handles:
            h.wait()

        def elem(e, carry2):
            inv = 1.0 / lens_v[e, pl.ds(0, _LANES)]
            cmean = []
            for k in range(_DC):
                s = ctx_rows[e * _L, pl.ds(k * _LANES, _LANES)]
                for j in range(1, _L):
                    s = s + ctx_rows[e * _L + j, pl.ds(k * _LANES, _LANES)]
                cmean.append(s * inv)
            for r in range(_T):
                v = cmean[0] * tgt_rows[e * _T + r, pl.ds(0, _LANES)]
                for k in range(1, _DC):
                    v = v + cmean[k] * tgt_rows[e * _T + r,
                                                pl.ds(k * _LANES, _LANES)]
                y_v[e * _T + r, pl.ds(0, _LANES)] = v
            return carry2

        lax.fori_loop(0, _CH, elem, 0)
        pltpu.sync_copy(y_v, y_hbm.at[pl.ds(pl.multiple_of(cb * _T, 8),
                                            _CH * _T)])
        return carry

    lax.fori_loop(0, _NCHUNK, chunk_body, 0)


def _loss_tc_body(y2_ref, o_ref):
    y2 = y2_ref[...]
    seg = (lax.broadcasted_iota(jnp.int32, (_Y2COLS, _LANES), 0) // _LANES
           == lax.broadcasted_iota(jnp.int32, (_Y2COLS, _LANES), 1))
    x = jnp.dot(y2, seg.astype(jnp.float32),
                preferred_element_type=jnp.float32)
    yr = (lax.broadcasted_iota(jnp.int32, (_Y2ROWS, _LANES), 0) * _LANES
          + lax.broadcasted_iota(jnp.int32, (_Y2ROWS, _LANES), 1))
    sgn = jnp.where(yr % _T == 0, 1.0, -1.0)
    terms = -jax.nn.log_sigmoid(sgn * jnp.clip(x, -10.0, 10.0))
    o_ref[...] = jnp.sum(terms)[None, None]


@jax.jit
def _cbow(ctx_flat, tgt_flat, lens_rep, emb0_weight, emb1_weight):
    mesh = plsc.VectorSubcoreMesh(core_axis_name="c", subcore_axis_name="s")
    y = pl.kernel(
        _cbow_sc_body,
        mesh=mesh,
        compiler_params=pltpu.CompilerParams(use_tc_tiling_on_sc=False),
        out_type=jax.ShapeDtypeStruct((_YROWS, _LANES), jnp.float32),
        scratch_types=[
            pltpu.VMEM((_CH * _L,), jnp.int32),
            pltpu.VMEM((_CH * _T,), jnp.int32),
            pltpu.VMEM((_CH, _LANES), jnp.float32),
            pltpu.VMEM((_CH * _L, _DIM), jnp.float32),
            pltpu.VMEM((_CH * _T, _DIM), jnp.float32),
            pltpu.VMEM((_CH * _T, _LANES), jnp.float32),
            pltpu.SemaphoreType.DMA,
            pltpu.SemaphoreType.DMA,
        ],
    )(ctx_flat, tgt_flat, lens_rep, emb0_weight, emb1_weight)
    o = pl.pallas_call(
        _loss_tc_body,
        out_shape=jax.ShapeDtypeStruct((1, 1), jnp.float32),
    )(y.reshape(_Y2ROWS, _Y2COLS))
    return o[0, 0]


def kernel(word_idx, ctx_inds, ctx_lens, neg_inds, emb0_weight, emb1_weight):
    ctx_flat = ctx_inds.astype(jnp.int32).reshape(-1)
    tgt_flat = jnp.concatenate(
        [word_idx[:, None], neg_inds], axis=1).astype(jnp.int32).reshape(-1)
    lens_rep = jnp.broadcast_to(
        ctx_lens.astype(jnp.float32)[:, None], (_B, _LANES))
    return _cbow(ctx_flat, tgt_flat, lens_rep, emb0_weight, emb1_weight)


# speedup vs baseline: 2.9919x; 1.0848x over previous
import jax
import jax.numpy as jnp
from jax import lax
from jax.experimental import pallas as pl
from jax.experimental.pallas import tpu as pltpu
from jax.experimental.pallas import tpu_sc as plsc

_VOCAB = 100000
_DIM = 64
_B = 16384
_L = 10
_NEG = 5
_T = _NEG + 1
_NC = 2
_NS = 16
_NW = _NC * _NS
_BPW = _B // _NW
_CH = 64
_NCHUNK = _BPW // _CH
_LANES = 16
_DC = _DIM // _LANES
_YROWS = _T * _B
_Y2COLS = 256
_Y2ROWS = _YROWS * _LANES // _Y2COLS
_BROWS = _B * _LANES // _Y2COLS


def _cbow_sc_body(ctx_idx_hbm, word_idx_hbm, neg_idx_hbm, emb0_hbm, emb1_hbm,
                  y_hbm,
                  ctx_idx, word_idx, neg_idx, ctx_rows, word_rows, neg_rows,
                  y_v, sem_g):
    wid = lax.axis_index("s") * _NC + lax.axis_index("c")
    base = wid * _BPW

    def chunk_body(c, carry):
        cb = base + c * _CH
        pltpu.sync_copy(
            ctx_idx_hbm.at[pl.ds(pl.multiple_of(cb * _L, 8), _CH * _L)],
            ctx_idx)
        pltpu.sync_copy(
            word_idx_hbm.at[pl.ds(pl.multiple_of(cb, 8), _CH)], word_idx)
        pltpu.sync_copy(
            neg_idx_hbm.at[pl.ds(pl.multiple_of(cb * _NEG, 8), _CH * _NEG)],
            neg_idx)
        handles = []
        for j in range(_CH * _L // 128):
            handles.append(pltpu.async_copy(
                emb0_hbm.at[ctx_idx.at[pl.ds(j * 128, 128)]],
                ctx_rows.at[pl.ds(j * 128, 128)], sem_g))
        handles.append(pltpu.async_copy(
            emb1_hbm.at[word_idx], word_rows, sem_g))
        for j in range(_CH * _NEG // 64):
            handles.append(pltpu.async_copy(
                emb1_hbm.at[neg_idx.at[pl.ds(j * 64, 64)]],
                neg_rows.at[pl.ds(j * 64, 64)], sem_g))
        for h in handles:
            h.wait()

        def elem(e, carry2):
            csum = []
            for k in range(_DC):
                s = ctx_rows[e * _L, pl.ds(k * _LANES, _LANES)]
                for j in range(1, _L):
                    s = s + ctx_rows[e * _L + j, pl.ds(k * _LANES, _LANES)]
                csum.append(s)
            v = csum[0] * word_rows[e, pl.ds(0, _LANES)]
            for k in range(1, _DC):
                v = v + csum[k] * word_rows[e, pl.ds(k * _LANES, _LANES)]
            y_v[0, e, pl.ds(0, _LANES)] = v
            for r in range(_NEG):
                v = csum[0] * neg_rows[e * _NEG + r, pl.ds(0, _LANES)]
                for k in range(1, _DC):
                    v = v + csum[k] * neg_rows[e * _NEG + r,
                                               pl.ds(k * _LANES, _LANES)]
                y_v[1 + r, e, pl.ds(0, _LANES)] = v
            return carry2

        lax.fori_loop(0, _CH, elem, 0)
        for r in range(_T):
            pltpu.sync_copy(
                y_v.at[r],
                y_hbm.at[pl.ds(pl.multiple_of(r * _B + cb, 8), _CH)])
        return carry

    lax.fori_loop(0, _NCHUNK, chunk_body, 0)


def _loss_tc_body(y2_ref, lens_ref, o_ref):
    y2 = y2_ref[...]
    seg = (lax.broadcasted_iota(jnp.int32, (_Y2COLS, _LANES), 0) // _LANES
           == lax.broadcasted_iota(jnp.int32, (_Y2COLS, _LANES), 1))
    x = jnp.dot(y2, seg.astype(jnp.float32),
                preferred_element_type=jnp.float32)
    x3 = x.reshape(_T, _BROWS, _LANES) / lens_ref[...][None, :, :]
    sgn = jnp.where(
        lax.broadcasted_iota(jnp.int32, (_T, _BROWS, _LANES), 0) == 0,
        1.0, -1.0)
    terms = -jax.nn.log_sigmoid(sgn * jnp.clip(x3, -10.0, 10.0))
    o_ref[...] = jnp.sum(terms)[None, None]


@jax.jit
def _cbow(ctx_flat, word_idx, neg_flat, lens2, emb0_weight, emb1_weight):
    mesh = plsc.VectorSubcoreMesh(core_axis_name="c", subcore_axis_name="s")
    y = pl.kernel(
        _cbow_sc_body,
        mesh=mesh,
        compiler_params=pltpu.CompilerParams(use_tc_tiling_on_sc=False),
        out_type=jax.ShapeDtypeStruct((_YROWS, _LANES), jnp.float32),
        scratch_types=[
            pltpu.VMEM((_CH * _L,), jnp.int32),
            pltpu.VMEM((_CH,), jnp.int32),
            pltpu.VMEM((_CH * _NEG,), jnp.int32),
            pltpu.VMEM((_CH * _L, _DIM), jnp.float32),
            pltpu.VMEM((_CH, _DIM), jnp.float32),
            pltpu.VMEM((_CH * _NEG, _DIM), jnp.float32),
            pltpu.VMEM((_T, _CH, _LANES), jnp.float32),
            pltpu.SemaphoreType.DMA,
        ],
    )(ctx_flat, word_idx, neg_flat, emb0_weight, emb1_weight)
    o = pl.pallas_call(
        _loss_tc_body,
        out_shape=jax.ShapeDtypeStruct((1, 1), jnp.float32),
    )(y.reshape(_Y2ROWS, _Y2COLS), lens2)
    return o[0, 0]


def kernel(word_idx, ctx_inds, ctx_lens, neg_inds, emb0_weight, emb1_weight):
    ctx_flat = ctx_inds.astype(jnp.int32).reshape(-1)
    neg_flat = neg_inds.astype(jnp.int32).reshape(-1)
    lens2 = ctx_lens.astype(jnp.float32).reshape(_BROWS, _LANES)
    return _cbow(ctx_flat, word_idx.astype(jnp.int32), neg_flat, lens2,
                 emb0_weight, emb1_weight)
